# Initial kernel scaffold; baseline (speedup 1.0000x reference)
import jax
import jax.numpy as jnp
from jax.experimental import pallas as pl


def kernel(x, edge_index, edge_attr, batch, atom_emb, bond_emb, conv_W1, conv_b1, conv_W2, conv_b2, conv_eps, fc_W, fc_b):
    raise NotImplementedError("write your pallas kernel here")


# speedup vs baseline: 6.8244x; 6.8244x over previous
import functools

import jax
import jax.numpy as jnp
from jax import lax
from jax.experimental import pallas as pl
from jax.experimental.pallas import tpu as pltpu
from jax.experimental.pallas import tpu_sc as plsc

N = 10000
E = 320000
HID = 128
NLAYERS = 4
NGRAPHS = 100

NPAD = 10240
BLK = 512
NBLK = NPAD // BLK

NC = 2
NS = 16
NW = NC * NS
CH = 128
EPW = 10112
NCHUNK = EPW // CH
EPAD = NW * EPW
RPT = NPAD // NS

_f32 = jnp.float32


def _sc_agg_body(g_hbm, cidx_hbm, dst_hbm, zeros_hbm, out_hbm,
                 cidx_v, dst_v, rows_v, zrow_v, agg_sh, sem):
    c = lax.axis_index("c")
    s = lax.axis_index("s")
    wid = s * NC + c
    pltpu.sync_copy(zeros_hbm, zrow_v)
    for j in range(RPT // CH):
        pltpu.sync_copy(zrow_v, agg_sh.at[pl.ds(s * RPT + j * CH, CH)])
    plsc.subcore_barrier()
    base = wid * EPW

    def chunk(i, carry):
        off = base + i * CH
        pltpu.sync_copy(cidx_hbm.at[pl.ds(off, CH)], cidx_v)
        pltpu.sync_copy(dst_hbm.at[pl.ds(off, CH)], dst_v)
        pltpu.async_copy(g_hbm.at[cidx_v], rows_v, sem).wait()
        pltpu.sync_copy(rows_v, agg_sh.at[dst_v], add=True)
        return carry

    lax.fori_loop(0, NCHUNK, chunk, 0)
    plsc.subcore_barrier()
    for j in range(RPT // CH):
        r = s * RPT + j * CH
        pltpu.sync_copy(agg_sh.at[pl.ds(r, CH)], rows_v)
        pltpu.sync_copy(rows_v, out_hbm.at[pl.ds(c * NPAD + r, CH)])


@functools.cache
def _build_sc_agg():
    return pl.kernel(
        _sc_agg_body,
        out_type=jax.ShapeDtypeStruct((NC * NPAD, HID), _f32),
        mesh=plsc.VectorSubcoreMesh(core_axis_name="c", subcore_axis_name="s",
                                    num_cores=NC, num_subcores=NS),
        scratch_types=[
            pltpu.VMEM((CH,), jnp.int32),
            pltpu.VMEM((CH,), jnp.int32),
            pltpu.VMEM((CH, HID), _f32),
            pltpu.VMEM((CH, HID), _f32),
            pltpu.VMEM_SHARED((NPAD, HID), _f32),
            pltpu.SemaphoreType.DMA,
        ],
    )


def _sc_agg(g, cidx_p, dst_p, zeros_c):
    return _build_sc_agg()(g.reshape(4 * NPAD, HID), cidx_p, dst_p, zeros_c)


def _encode_body(xv_ref, atom_ref, bond_ref, h_ref, g_ref):
    ids = xv_ref[...]
    oh = (ids == lax.broadcasted_iota(jnp.int32, (BLK, 32), 1).astype(_f32)
          ).astype(_f32)
    h = jnp.dot(oh, atom_ref[...], preferred_element_type=_f32)
    h_ref[...] = h
    for t in range(4):
        g_ref[t] = jnp.maximum(h + bond_ref[t], 0.0)


def _encode(xvf, atom_p, bond_p):
    return pl.pallas_call(
        _encode_body,
        grid=(NBLK,),
        in_specs=[
            pl.BlockSpec((BLK, 1), lambda b: (b, 0)),
            pl.BlockSpec((32, HID), lambda b: (0, 0)),
            pl.BlockSpec((8, HID), lambda b: (0, 0)),
        ],
        out_specs=[
            pl.BlockSpec((BLK, HID), lambda b: (b, 0)),
            pl.BlockSpec((4, BLK, HID), lambda b: (0, b, 0)),
        ],
        out_shape=[
            jax.ShapeDtypeStruct((NPAD, HID), _f32),
            jax.ShapeDtypeStruct((4, NPAD, HID), _f32),
        ],
    )(xvf, atom_p, bond_p)


def _layer_body(h_ref, agg_ref, w1_ref, b1_ref, w2_ref, b2_ref, eps_ref,
                bond_ref, hout_ref, g_ref):
    pre = h_ref[...] * eps_ref[...] + agg_ref[0] + agg_ref[1]
    t = jnp.maximum(
        jnp.dot(pre, w1_ref[...], preferred_element_type=_f32) + b1_ref[...],
        0.0)
    out = jnp.dot(t, w2_ref[...], preferred_element_type=_f32) + b2_ref[...]
    hout_ref[...] = out
    for k in range(4):
        g_ref[k] = jnp.maximum(out + bond_ref[k], 0.0)


def _layer(h, agg2, w1, b1, w2, b2, epsr, bond_p):
    return pl.pallas_call(
        _layer_body,
        grid=(NBLK,),
        in_specs=[
            pl.BlockSpec((BLK, HID), lambda b: (b, 0)),
            pl.BlockSpec((2, BLK, HID), lambda b: (0, b, 0)),
            pl.BlockSpec((HID, HID), lambda b: (0, 0)),
            pl.BlockSpec((1, HID), lambda b: (0, 0)),
            pl.BlockSpec((HID, HID), lambda b: (0, 0)),
            pl.BlockSpec((1, HID), lambda b: (0, 0)),
            pl.BlockSpec((1, HID), lambda b: (0, 0)),
            pl.BlockSpec((8, HID), lambda b: (0, 0)),
        ],
        out_specs=[
            pl.BlockSpec((BLK, HID), lambda b: (b, 0)),
            pl.BlockSpec((4, BLK, HID), lambda b: (0, b, 0)),
        ],
        out_shape=[
            jax.ShapeDtypeStruct((NPAD, HID), _f32),
            jax.ShapeDtypeStruct((4, NPAD, HID), _f32),
        ],
    )(h, agg2, w1, b1, w2, b2, epsr, bond_p)


def _final_body(h_ref, agg_ref, w1_ref, b1_ref, w2_ref, b2_ref, eps_ref,
                bat_ref, fcw_ref, fcb_ref, pred_ref, sums, cnt):
    b = pl.program_id(0)
    pre = h_ref[...] * eps_ref[...] + agg_ref[0] + agg_ref[1]
    t = jnp.maximum(
        jnp.dot(pre, w1_ref[...], preferred_element_type=_f32) + b1_ref[...],
        0.0)
    out = jnp.dot(t, w2_ref[...], preferred_element_type=_f32) + b2_ref[...]
    oh = (bat_ref[...] ==
          lax.broadcasted_iota(jnp.int32, (BLK, HID), 1).astype(_f32)
          ).astype(_f32)
    part = lax.dot_general(oh, out, (((0,), (0,)), ((), ())),
                           preferred_element_type=_f32)
    cpart = lax.dot_general(oh, jnp.ones((BLK, HID), _f32),
                            (((0,), (0,)), ((), ())),
                            preferred_element_type=_f32)

    @pl.when(b == 0)
    def _():
        sums[...] = jnp.zeros_like(sums)
        cnt[...] = jnp.zeros_like(cnt)

    sums[...] += part
    cnt[...] += cpart

    @pl.when(b == NBLK - 1)
    def _():
        ge = sums[...] / jnp.maximum(cnt[...], 1.0)
        pred_ref[...] = (jnp.dot(ge, fcw_ref[...], preferred_element_type=_f32)
                         + fcb_ref[...])


def _final(h, agg2, w1, b1, w2, b2, epsr, batf, fcw, fcb):
    return pl.pallas_call(
        _final_body,
        grid=(NBLK,),
        in_specs=[
            pl.BlockSpec((BLK, HID), lambda b: (b, 0)),
            pl.BlockSpec((2, BLK, HID), lambda b: (0, b, 0)),
            pl.BlockSpec((HID, HID), lambda b: (0, 0)),
            pl.BlockSpec((1, HID), lambda b: (0, 0)),
            pl.BlockSpec((HID, HID), lambda b: (0, 0)),
            pl.BlockSpec((1, HID), lambda b: (0, 0)),
            pl.BlockSpec((1, HID), lambda b: (0, 0)),
            pl.BlockSpec((BLK, 1), lambda b: (b, 0)),
            pl.BlockSpec((HID, HID), lambda b: (0, 0)),
            pl.BlockSpec((1, HID), lambda b: (0, 0)),
        ],
        out_specs=pl.BlockSpec((HID, HID), lambda b: (0, 0)),
        out_shape=jax.ShapeDtypeStruct((HID, HID), _f32),
        scratch_shapes=[
            pltpu.VMEM((HID, HID), _f32),
            pltpu.VMEM((HID, HID), _f32),
        ],
    )(h, agg2, w1, b1, w2, b2, epsr, batf, fcw, fcb)


def kernel(x, edge_index, edge_attr, batch, atom_emb, bond_emb,
           conv_W1, conv_b1, conv_W2, conv_b2, conv_eps, fc_W, fc_b):
    xvf = jnp.concatenate(
        [x[:, 0].astype(_f32), jnp.full((NPAD - N,), -1.0, _f32)]
    ).reshape(NPAD, 1)
    src = edge_index[0].astype(jnp.int32)
    dst = edge_index[1].astype(jnp.int32)
    attr = edge_attr.astype(jnp.int32)
    padr = N + (jnp.arange(EPAD - E, dtype=jnp.int32) % (NPAD - N))
    cidx_p = jnp.concatenate([attr * NPAD + src, 3 * NPAD + padr])
    dst_p = jnp.concatenate([dst, padr])
    batf = jnp.concatenate(
        [batch.astype(_f32), jnp.full((NPAD - N,), 127.0, _f32)]
    ).reshape(NPAD, 1)
    atom_p = jnp.zeros((32, HID), _f32).at[:28].set(atom_emb)
    bond_p = jnp.zeros((8, HID), _f32).at[:4].set(bond_emb)
    zeros_c = jnp.zeros((CH, HID), _f32)
    b1 = conv_b1.reshape(NLAYERS, 1, HID)
    b2 = conv_b2.reshape(NLAYERS, 1, HID)
    epsr = jnp.broadcast_to((1.0 + conv_eps)[:, None, None],
                            (NLAYERS, 1, HID)).astype(_f32)
    fcb = fc_b.reshape(1, HID)

    h, g = _encode(xvf, atom_p, bond_p)
    for i in range(NLAYERS - 1):
        agg2 = _sc_agg(g, cidx_p, dst_p, zeros_c).reshape(NC, NPAD, HID)
        h, g = _layer(h, agg2, conv_W1[i], b1[i], conv_W2[i], b2[i],
                      epsr[i], bond_p)
    agg2 = _sc_agg(g, cidx_p, dst_p, zeros_c).reshape(NC, NPAD, HID)
    pred = _final(h, agg2, conv_W1[3], b1[3], conv_W2[3], b2[3],
                  epsr[3], batf, fc_W, fcb)
    return pred[:NGRAPHS]


# speedup vs baseline: 11.4540x; 1.6784x over previous
import functools

import jax
import jax.numpy as jnp
from jax import lax
from jax.experimental import pallas as pl
from jax.experimental.pallas import tpu as pltpu
from jax.experimental.pallas import tpu_sc as plsc

N = 10000
E = 320000
HID = 128
NLAYERS = 4
NGRAPHS = 100

NPAD = 10240
BLK = 512
NBLK = NPAD // BLK

NC = 2
NS = 16
NW = NC * NS
CH = 128
NCHUNK = 80
EPW = NCHUNK * CH
EPAD = NW * EPW
RPT = NPAD // NS

_f32 = jnp.float32


def _sc_agg_body(g_hbm, cidx_hbm, dst_hbm, zeros_hbm, out_hbm,
                 idx_tab, dst_a, dst_b, rows_a, rows_b, agg_sh,
                 gsem_a, gsem_b, dsem_a, dsem_b):
    c = lax.axis_index("c")
    s = lax.axis_index("s")
    wid = s * NC + c
    pltpu.sync_copy(cidx_hbm.at[pl.ds(wid * NCHUNK, NCHUNK)], idx_tab)
    pltpu.sync_copy(zeros_hbm, rows_a)
    for j in range(RPT // CH):
        pltpu.sync_copy(rows_a, agg_sh.at[pl.ds(s * RPT + j * CH, CH)])
    plsc.subcore_barrier()
    ebase = wid * EPW

    pltpu.async_copy(g_hbm.at[idx_tab.at[0]], rows_a, gsem_a)
    pltpu.async_copy(dst_hbm.at[pl.ds(ebase, CH)], dst_a, dsem_a)

    def pair(k, carry):
        i = 2 * k
        pltpu.make_async_copy(g_hbm.at[idx_tab.at[i]], rows_a, gsem_a).wait()
        pltpu.make_async_copy(dst_hbm.at[pl.ds(ebase, CH)], dst_a, dsem_a).wait()
        pltpu.async_copy(g_hbm.at[idx_tab.at[i + 1]], rows_b, gsem_b)
        pltpu.async_copy(dst_hbm.at[pl.ds(ebase + (i + 1) * CH, CH)], dst_b,
                         dsem_b)
        pltpu.sync_copy(rows_a, agg_sh.at[dst_a], add=True)
        pltpu.make_async_copy(g_hbm.at[idx_tab.at[i]], rows_b, gsem_b).wait()
        pltpu.make_async_copy(dst_hbm.at[pl.ds(ebase, CH)], dst_b, dsem_b).wait()

        @pl.when(k < NCHUNK // 2 - 1)
        def _():
            pltpu.async_copy(g_hbm.at[idx_tab.at[i + 2]], rows_a, gsem_a)
            pltpu.async_copy(dst_hbm.at[pl.ds(ebase + (i + 2) * CH, CH)],
                             dst_a, dsem_a)

        pltpu.sync_copy(rows_b, agg_sh.at[dst_b], add=True)
        return carry

    lax.fori_loop(0, NCHUNK // 2, pair, 0)
    plsc.subcore_barrier()
    for j in range(RPT // CH):
        r = s * RPT + j * CH
        buf = rows_a if j % 2 == 0 else rows_b
        pltpu.sync_copy(agg_sh.at[pl.ds(r, CH)], buf)
        pltpu.sync_copy(buf, out_hbm.at[pl.ds(c * NPAD + r, CH)])


@functools.cache
def _build_sc_agg():
    return pl.kernel(
        _sc_agg_body,
        out_type=jax.ShapeDtypeStruct((NC * NPAD, HID), _f32),
        mesh=plsc.VectorSubcoreMesh(core_axis_name="c", subcore_axis_name="s",
                                    num_cores=NC, num_subcores=NS),
        scratch_types=[
            pltpu.VMEM((NCHUNK, CH), jnp.int32),
            pltpu.VMEM((CH,), jnp.int32),
            pltpu.VMEM((CH,), jnp.int32),
            pltpu.VMEM((CH, HID), _f32),
            pltpu.VMEM((CH, HID), _f32),
            pltpu.VMEM_SHARED((NPAD, HID), _f32),
            pltpu.SemaphoreType.DMA,
            pltpu.SemaphoreType.DMA,
            pltpu.SemaphoreType.DMA,
            pltpu.SemaphoreType.DMA,
        ],
    )


def _sc_agg(g, cidx_p, dst_p, zeros_c):
    return _build_sc_agg()(g.reshape(4 * NPAD, HID),
                           cidx_p.reshape(NW * NCHUNK, CH), dst_p, zeros_c)


def _encode_body(xv_ref, atom_ref, bond_ref, h_ref, g_ref):
    ids = xv_ref[...]
    oh = (ids == lax.broadcasted_iota(jnp.int32, (BLK, 32), 1).astype(_f32)
          ).astype(_f32)
    h = jnp.dot(oh, atom_ref[...], preferred_element_type=_f32)
    h_ref[...] = h
    for t in range(4):
        g_ref[t] = jnp.maximum(h + bond_ref[t], 0.0)


def _encode(xvf, atom_p, bond_p):
    return pl.pallas_call(
        _encode_body,
        grid=(NBLK,),
        in_specs=[
            pl.BlockSpec((BLK, 1), lambda b: (b, 0)),
            pl.BlockSpec((32, HID), lambda b: (0, 0)),
            pl.BlockSpec((8, HID), lambda b: (0, 0)),
        ],
        out_specs=[
            pl.BlockSpec((BLK, HID), lambda b: (b, 0)),
            pl.BlockSpec((4, BLK, HID), lambda b: (0, b, 0)),
        ],
        out_shape=[
            jax.ShapeDtypeStruct((NPAD, HID), _f32),
            jax.ShapeDtypeStruct((4, NPAD, HID), _f32),
        ],
    )(xvf, atom_p, bond_p)


def _layer_body(h_ref, agg_ref, w1_ref, b1_ref, w2_ref, b2_ref, eps_ref,
                bond_ref, hout_ref, g_ref):
    pre = h_ref[...] * eps_ref[...] + agg_ref[0] + agg_ref[1]
    t = jnp.maximum(
        jnp.dot(pre, w1_ref[...], preferred_element_type=_f32) + b1_ref[...],
        0.0)
    out = jnp.dot(t, w2_ref[...], preferred_element_type=_f32) + b2_ref[...]
    hout_ref[...] = out
    for k in range(4):
        g_ref[k] = jnp.maximum(out + bond_ref[k], 0.0)


def _layer(h, agg2, w1, b1, w2, b2, epsr, bond_p):
    return pl.pallas_call(
        _layer_body,
        grid=(NBLK,),
        in_specs=[
            pl.BlockSpec((BLK, HID), lambda b: (b, 0)),
            pl.BlockSpec((2, BLK, HID), lambda b: (0, b, 0)),
            pl.BlockSpec((HID, HID), lambda b: (0, 0)),
            pl.BlockSpec((1, HID), lambda b: (0, 0)),
            pl.BlockSpec((HID, HID), lambda b: (0, 0)),
            pl.BlockSpec((1, HID), lambda b: (0, 0)),
            pl.BlockSpec((1, HID), lambda b: (0, 0)),
            pl.BlockSpec((8, HID), lambda b: (0, 0)),
        ],
        out_specs=[
            pl.BlockSpec((BLK, HID), lambda b: (b, 0)),
            pl.BlockSpec((4, BLK, HID), lambda b: (0, b, 0)),
        ],
        out_shape=[
            jax.ShapeDtypeStruct((NPAD, HID), _f32),
            jax.ShapeDtypeStruct((4, NPAD, HID), _f32),
        ],
    )(h, agg2, w1, b1, w2, b2, epsr, bond_p)


def _final_body(h_ref, agg_ref, w1_ref, b1_ref, w2_ref, b2_ref, eps_ref,
                bat_ref, fcw_ref, fcb_ref, pred_ref, sums, cnt):
    b = pl.program_id(0)
    pre = h_ref[...] * eps_ref[...] + agg_ref[0] + agg_ref[1]
    t = jnp.maximum(
        jnp.dot(pre, w1_ref[...], preferred_element_type=_f32) + b1_ref[...],
        0.0)
    out = jnp.dot(t, w2_ref[...], preferred_element_type=_f32) + b2_ref[...]
    oh = (bat_ref[...] ==
          lax.broadcasted_iota(jnp.int32, (BLK, HID), 1).astype(_f32)
          ).astype(_f32)
    part = lax.dot_general(oh, out, (((0,), (0,)), ((), ())),
                           preferred_element_type=_f32)
    cpart = lax.dot_general(oh, jnp.ones((BLK, HID), _f32),
                            (((0,), (0,)), ((), ())),
                            preferred_element_type=_f32)

    @pl.when(b == 0)
    def _():
        sums[...] = jnp.zeros_like(sums)
        cnt[...] = jnp.zeros_like(cnt)

    sums[...] += part
    cnt[...] += cpart

    @pl.when(b == NBLK - 1)
    def _():
        ge = sums[...] / jnp.maximum(cnt[...], 1.0)
        pred_ref[...] = (jnp.dot(ge, fcw_ref[...], preferred_element_type=_f32)
                         + fcb_ref[...])


def _final(h, agg2, w1, b1, w2, b2, epsr, batf, fcw, fcb):
    return pl.pallas_call(
        _final_body,
        grid=(NBLK,),
        in_specs=[
            pl.BlockSpec((BLK, HID), lambda b: (b, 0)),
            pl.BlockSpec((2, BLK, HID), lambda b: (0, b, 0)),
            pl.BlockSpec((HID, HID), lambda b: (0, 0)),
            pl.BlockSpec((1, HID), lambda b: (0, 0)),
            pl.BlockSpec((HID, HID), lambda b: (0, 0)),
            pl.BlockSpec((1, HID), lambda b: (0, 0)),
            pl.BlockSpec((1, HID), lambda b: (0, 0)),
            pl.BlockSpec((BLK, 1), lambda b: (b, 0)),
            pl.BlockSpec((HID, HID), lambda b: (0, 0)),
            pl.BlockSpec((1, HID), lambda b: (0, 0)),
        ],
        out_specs=pl.BlockSpec((HID, HID), lambda b: (0, 0)),
        out_shape=jax.ShapeDtypeStruct((HID, HID), _f32),
        scratch_shapes=[
            pltpu.VMEM((HID, HID), _f32),
            pltpu.VMEM((HID, HID), _f32),
        ],
    )(h, agg2, w1, b1, w2, b2, epsr, batf, fcw, fcb)


def kernel(x, edge_index, edge_attr, batch, atom_emb, bond_emb,
           conv_W1, conv_b1, conv_W2, conv_b2, conv_eps, fc_W, fc_b):
    xvf = jnp.concatenate(
        [x[:, 0].astype(_f32), jnp.full((NPAD - N,), -1.0, _f32)]
    ).reshape(NPAD, 1)
    src = edge_index[0].astype(jnp.int32)
    dst = edge_index[1].astype(jnp.int32)
    attr = edge_attr.astype(jnp.int32)
    padr = N + (jnp.arange(EPAD - E, dtype=jnp.int32) % (NPAD - N))
    cidx_p = jnp.concatenate([attr * NPAD + src, 3 * NPAD + padr])
    dst_p = jnp.concatenate([dst, padr])
    batf = jnp.concatenate(
        [batch.astype(_f32), jnp.full((NPAD - N,), 127.0, _f32)]
    ).reshape(NPAD, 1)
    atom_p = jnp.zeros((32, HID), _f32).at[:28].set(atom_emb)
    bond_p = jnp.zeros((8, HID), _f32).at[:4].set(bond_emb)
    zeros_c = jnp.zeros((CH, HID), _f32)
    b1 = conv_b1.reshape(NLAYERS, 1, HID)
    b2 = conv_b2.reshape(NLAYERS, 1, HID)
    epsr = jnp.broadcast_to((1.0 + conv_eps)[:, None, None],
                            (NLAYERS, 1, HID)).astype(_f32)
    fcb = fc_b.reshape(1, HID)

    h, g = _encode(xvf, atom_p, bond_p)
    for i in range(NLAYERS - 1):
        agg2 = _sc_agg(g, cidx_p, dst_p, zeros_c).reshape(NC, NPAD, HID)
        h, g = _layer(h, agg2, conv_W1[i], b1[i], conv_W2[i], b2[i],
                      epsr[i], bond_p)
    agg2 = _sc_agg(g, cidx_p, dst_p, zeros_c).reshape(NC, NPAD, HID)
    pred = _final(h, agg2, conv_W1[3], b1[3], conv_W2[3], b2[3],
                  epsr[3], batf, fc_W, fcb)
    return pred[:NGRAPHS]


# speedup vs baseline: 12.1739x; 1.0628x over previous
import functools

import jax
import jax.numpy as jnp
from jax import lax
from jax.experimental import pallas as pl
from jax.experimental.pallas import tpu as pltpu
from jax.experimental.pallas import tpu_sc as plsc

N = 10000
E = 320000
HID = 128
NLAYERS = 4
NGRAPHS = 100

NPAD = 10240
BLK = 512
NBLK = NPAD // BLK

NC = 2
NS = 16
NW = NC * NS
CH = 64
NCHUNK = 160
EPW = NCHUNK * CH
EPAD = NW * EPW
RPT = NPAD // NS

_f32 = jnp.float32


def _sc_agg_body(g_hbm, cidx_hbm, dst_hbm, zeros_hbm, out_hbm, *sc):
    rows = sc[0:4]
    cid = sc[4:8]
    dst = sc[8:12]
    agg_sh = sc[12]
    gsem = sc[13:17]
    isem = sc[17:21]
    dsem = sc[21:25]
    ssem = sc[25:29]
    c = lax.axis_index("c")
    s = lax.axis_index("s")
    wid = s * NC + c
    ebase = wid * EPW
    K = NCHUNK // 4

    def load_cidx(chunk, j):
        pltpu.async_copy(cidx_hbm.at[pl.ds(ebase + chunk * CH, CH)], cid[j],
                         isem[j])

    def load_dst(chunk, j):
        pltpu.async_copy(dst_hbm.at[pl.ds(ebase + chunk * CH, CH)], dst[j],
                         dsem[j])

    pltpu.sync_copy(zeros_hbm, rows[0])
    for j in range(RPT // CH):
        pltpu.sync_copy(rows[0], agg_sh.at[pl.ds(s * RPT + j * CH, CH)])
    plsc.subcore_barrier()

    for j in range(4):
        load_cidx(j, j)
    for j in range(2):
        load_dst(j, j)
    for j in range(2):
        pltpu.make_async_copy(cidx_hbm.at[pl.ds(0, CH)], cid[j], isem[j]).wait()
        pltpu.async_copy(g_hbm.at[cid[j]], rows[j], gsem[j])

    def body(k, carry):
        for j in range(4):
            jp = (j + 2) % 4
            i = 4 * k + j
            pltpu.make_async_copy(g_hbm.at[cid[j]], rows[j], gsem[j]).wait()

            @pl.when(k < K - 1)
            def _(i=i, j=j):
                load_cidx(i + 4, j)

            def wait_prev_scatter(jp=jp):
                pltpu.make_async_copy(rows[jp], agg_sh.at[dst[jp]],
                                      ssem[jp]).wait()
            if j < 2:
                pl.when(k > 0)(wait_prev_scatter)
            else:
                wait_prev_scatter()

            def issue_next(i=i, jp=jp):
                pltpu.make_async_copy(cidx_hbm.at[pl.ds(0, CH)], cid[jp],
                                      isem[jp]).wait()
                pltpu.async_copy(g_hbm.at[cid[jp]], rows[jp], gsem[jp])
                load_dst(i + 2, jp)
            if j < 2:
                issue_next()
            else:
                pl.when(k < K - 1)(issue_next)

            pltpu.make_async_copy(dst_hbm.at[pl.ds(0, CH)], dst[j],
                                  dsem[j]).wait()
            pltpu.async_copy(rows[j], agg_sh.at[dst[j]], ssem[j], add=True)
        return carry

    lax.fori_loop(0, K, body, 0)
    for j in (2, 3):
        pltpu.make_async_copy(rows[j], agg_sh.at[dst[j]], ssem[j]).wait()
    plsc.subcore_barrier()
    for j in range(RPT // CH):
        r = s * RPT + j * CH
        buf = rows[j % 4]
        pltpu.sync_copy(agg_sh.at[pl.ds(r, CH)], buf)
        pltpu.sync_copy(buf, out_hbm.at[pl.ds(c * NPAD + r, CH)])


@functools.cache
def _build_sc_agg():
    return pl.kernel(
        _sc_agg_body,
        out_type=jax.ShapeDtypeStruct((NC * NPAD, HID), _f32),
        mesh=plsc.VectorSubcoreMesh(core_axis_name="c", subcore_axis_name="s",
                                    num_cores=NC, num_subcores=NS),
        scratch_types=(
            [pltpu.VMEM((CH, HID), _f32)] * 4
            + [pltpu.VMEM((CH,), jnp.int32)] * 8
            + [pltpu.VMEM_SHARED((NPAD, HID), _f32)]
            + [pltpu.SemaphoreType.DMA] * 16
        ),
    )


def _sc_agg(g, cidx_p, dst_p, zeros_c):
    return _build_sc_agg()(g.reshape(4 * NPAD, HID), cidx_p, dst_p, zeros_c)


def _encode_body(xv_ref, atom_ref, bond_ref, h_ref, g_ref):
    ids = xv_ref[...]
    oh = (ids == lax.broadcasted_iota(jnp.int32, (BLK, 32), 1).astype(_f32)
          ).astype(_f32)
    h = jnp.dot(oh, atom_ref[...], preferred_element_type=_f32)
    h_ref[...] = h
    for t in range(4):
        g_ref[t] = jnp.maximum(h + bond_ref[t], 0.0)


def _encode(xvf, atom_p, bond_p):
    return pl.pallas_call(
        _encode_body,
        grid=(NBLK,),
        in_specs=[
            pl.BlockSpec((BLK, 1), lambda b: (b, 0)),
            pl.BlockSpec((32, HID), lambda b: (0, 0)),
            pl.BlockSpec((8, HID), lambda b: (0, 0)),
        ],
        out_specs=[
            pl.BlockSpec((BLK, HID), lambda b: (b, 0)),
            pl.BlockSpec((4, BLK, HID), lambda b: (0, b, 0)),
        ],
        out_shape=[
            jax.ShapeDtypeStruct((NPAD, HID), _f32),
            jax.ShapeDtypeStruct((4, NPAD, HID), _f32),
        ],
    )(xvf, atom_p, bond_p)


def _layer_body(h_ref, agg_ref, w1_ref, b1_ref, w2_ref, b2_ref, eps_ref,
                bond_ref, hout_ref, g_ref):
    pre = h_ref[...] * eps_ref[...] + agg_ref[0] + agg_ref[1]
    t = jnp.maximum(
        jnp.dot(pre, w1_ref[...], preferred_element_type=_f32) + b1_ref[...],
        0.0)
    out = jnp.dot(t, w2_ref[...], preferred_element_type=_f32) + b2_ref[...]
    hout_ref[...] = out
    for k in range(4):
        g_ref[k] = jnp.maximum(out + bond_ref[k], 0.0)


def _layer(h, agg2, w1, b1, w2, b2, epsr, bond_p):
    return pl.pallas_call(
        _layer_body,
        grid=(NBLK,),
        in_specs=[
            pl.BlockSpec((BLK, HID), lambda b: (b, 0)),
            pl.BlockSpec((2, BLK, HID), lambda b: (0, b, 0)),
            pl.BlockSpec((HID, HID), lambda b: (0, 0)),
            pl.BlockSpec((1, HID), lambda b: (0, 0)),
            pl.BlockSpec((HID, HID), lambda b: (0, 0)),
            pl.BlockSpec((1, HID), lambda b: (0, 0)),
            pl.BlockSpec((1, HID), lambda b: (0, 0)),
            pl.BlockSpec((8, HID), lambda b: (0, 0)),
        ],
        out_specs=[
            pl.BlockSpec((BLK, HID), lambda b: (b, 0)),
            pl.BlockSpec((4, BLK, HID), lambda b: (0, b, 0)),
        ],
        out_shape=[
            jax.ShapeDtypeStruct((NPAD, HID), _f32),
            jax.ShapeDtypeStruct((4, NPAD, HID), _f32),
        ],
    )(h, agg2, w1, b1, w2, b2, epsr, bond_p)


def _final_body(h_ref, agg_ref, w1_ref, b1_ref, w2_ref, b2_ref, eps_ref,
                bat_ref, fcw_ref, fcb_ref, pred_ref, sums, cnt):
    b = pl.program_id(0)
    pre = h_ref[...] * eps_ref[...] + agg_ref[0] + agg_ref[1]
    t = jnp.maximum(
        jnp.dot(pre, w1_ref[...], preferred_element_type=_f32) + b1_ref[...],
        0.0)
    out = jnp.dot(t, w2_ref[...], preferred_element_type=_f32) + b2_ref[...]
    oh = (bat_ref[...] ==
          lax.broadcasted_iota(jnp.int32, (BLK, HID), 1).astype(_f32)
          ).astype(_f32)
    part = lax.dot_general(oh, out, (((0,), (0,)), ((), ())),
                           preferred_element_type=_f32)
    cpart = lax.dot_general(oh, jnp.ones((BLK, HID), _f32),
                            (((0,), (0,)), ((), ())),
                            preferred_element_type=_f32)

    @pl.when(b == 0)
    def _():
        sums[...] = jnp.zeros_like(sums)
        cnt[...] = jnp.zeros_like(cnt)

    sums[...] += part
    cnt[...] += cpart

    @pl.when(b == NBLK - 1)
    def _():
        ge = sums[...] / jnp.maximum(cnt[...], 1.0)
        pred_ref[...] = (jnp.dot(ge, fcw_ref[...], preferred_element_type=_f32)
                         + fcb_ref[...])


def _final(h, agg2, w1, b1, w2, b2, epsr, batf, fcw, fcb):
    return pl.pallas_call(
        _final_body,
        grid=(NBLK,),
        in_specs=[
            pl.BlockSpec((BLK, HID), lambda b: (b, 0)),
            pl.BlockSpec((2, BLK, HID), lambda b: (0, b, 0)),
            pl.BlockSpec((HID, HID), lambda b: (0, 0)),
            pl.BlockSpec((1, HID), lambda b: (0, 0)),
            pl.BlockSpec((HID, HID), lambda b: (0, 0)),
            pl.BlockSpec((1, HID), lambda b: (0, 0)),
            pl.BlockSpec((1, HID), lambda b: (0, 0)),
            pl.BlockSpec((BLK, 1), lambda b: (b, 0)),
            pl.BlockSpec((HID, HID), lambda b: (0, 0)),
            pl.BlockSpec((1, HID), lambda b: (0, 0)),
        ],
        out_specs=pl.BlockSpec((HID, HID), lambda b: (0, 0)),
        out_shape=jax.ShapeDtypeStruct((HID, HID), _f32),
        scratch_shapes=[
            pltpu.VMEM((HID, HID), _f32),
            pltpu.VMEM((HID, HID), _f32),
        ],
    )(h, agg2, w1, b1, w2, b2, epsr, batf, fcw, fcb)


def kernel(x, edge_index, edge_attr, batch, atom_emb, bond_emb,
           conv_W1, conv_b1, conv_W2, conv_b2, conv_eps, fc_W, fc_b):
    xvf = jnp.concatenate(
        [x[:, 0].astype(_f32), jnp.full((NPAD - N,), -1.0, _f32)]
    ).reshape(NPAD, 1)
    src = edge_index[0].astype(jnp.int32)
    dst = edge_index[1].astype(jnp.int32)
    attr = edge_attr.astype(jnp.int32)
    padr = N + (jnp.arange(EPAD - E, dtype=jnp.int32) % (NPAD - N))
    cidx_p = jnp.concatenate([attr * NPAD + src, 3 * NPAD + padr])
    dst_p = jnp.concatenate([dst, padr])
    batf = jnp.concatenate(
        [batch.astype(_f32), jnp.full((NPAD - N,), 127.0, _f32)]
    ).reshape(NPAD, 1)
    atom_p = jnp.zeros((32, HID), _f32).at[:28].set(atom_emb)
    bond_p = jnp.zeros((8, HID), _f32).at[:4].set(bond_emb)
    zeros_c = jnp.zeros((CH, HID), _f32)
    b1 = conv_b1.reshape(NLAYERS, 1, HID)
    b2 = conv_b2.reshape(NLAYERS, 1, HID)
    epsr = jnp.broadcast_to((1.0 + conv_eps)[:, None, None],
                            (NLAYERS, 1, HID)).astype(_f32)
    fcb = fc_b.reshape(1, HID)

    h, g = _encode(xvf, atom_p, bond_p)
    for i in range(NLAYERS - 1):
        agg2 = _sc_agg(g, cidx_p, dst_p, zeros_c).reshape(NC, NPAD, HID)
        h, g = _layer(h, agg2, conv_W1[i], b1[i], conv_W2[i], b2[i],
                      epsr[i], bond_p)
    agg2 = _sc_agg(g, cidx_p, dst_p, zeros_c).reshape(NC, NPAD, HID)
    pred = _final(h, agg2, conv_W1[3], b1[3], conv_W2[3], b2[3],
                  epsr[3], batf, fc_W, fcb)
    return pred[:NGRAPHS]


# speedup vs baseline: 13.0506x; 1.0720x over previous
import functools

import jax
import jax.numpy as jnp
from jax import lax
from jax.experimental import pallas as pl
from jax.experimental.pallas import tpu as pltpu
from jax.experimental.pallas import tpu_sc as plsc

N = 10000
E = 320000
HID = 128
NLAYERS = 4
NGRAPHS = 100

NPAD = 10240
BLK = 1024
NBLK = NPAD // BLK

NC = 2
NS = 16
NW = NC * NS
CH = 64
NCHUNK = 160
EPW = NCHUNK * CH
EPAD = NW * EPW
RPT = NPAD // NS

_f32 = jnp.float32


def _sc_agg_body(g_hbm, cidx_hbm, dst_hbm, zeros_hbm, out_hbm, *sc):
    rows = sc[0:4]
    cid = sc[4:8]
    dst = sc[8:12]
    agg_sh = sc[12]
    gsem = sc[13:17]
    isem = sc[17:21]
    dsem = sc[21:25]
    ssem = sc[25:29]
    c = lax.axis_index("c")
    s = lax.axis_index("s")
    wid = s * NC + c
    ebase = wid * EPW
    K = NCHUNK // 4

    def load_cidx(chunk, j):
        pltpu.async_copy(cidx_hbm.at[pl.ds(ebase + chunk * CH, CH)], cid[j],
                         isem[j])

    def load_dst(chunk, j):
        pltpu.async_copy(dst_hbm.at[pl.ds(ebase + chunk * CH, CH)], dst[j],
                         dsem[j])

    pltpu.sync_copy(zeros_hbm, rows[0])
    for j in range(RPT // CH):
        pltpu.async_copy(rows[0], agg_sh.at[pl.ds(s * RPT + j * CH, CH)],
                         gsem[j % 4])
    for j in range(RPT // CH):
        pltpu.make_async_copy(
            rows[0], agg_sh.at[pl.ds(s * RPT + (j % 4) * CH, CH)],
            gsem[j % 4]).wait()
    plsc.subcore_barrier()

    for j in range(4):
        load_cidx(j, j)
    for j in range(2):
        load_dst(j, j)
    for j in range(2):
        pltpu.make_async_copy(cidx_hbm.at[pl.ds(0, CH)], cid[j], isem[j]).wait()
        pltpu.async_copy(g_hbm.at[cid[j]], rows[j], gsem[j])

    def body(k, carry):
        for j in range(4):
            jp = (j + 2) % 4
            i = 4 * k + j
            pltpu.make_async_copy(g_hbm.at[cid[j]], rows[j], gsem[j]).wait()

            @pl.when(k < K - 1)
            def _(i=i, j=j):
                load_cidx(i + 4, j)

            def wait_prev_scatter(jp=jp):
                pltpu.make_async_copy(rows[jp], agg_sh.at[dst[jp]],
                                      ssem[jp]).wait()
            if j < 2:
                pl.when(k > 0)(wait_prev_scatter)
            else:
                wait_prev_scatter()

            def issue_next(i=i, jp=jp):
                pltpu.make_async_copy(cidx_hbm.at[pl.ds(0, CH)], cid[jp],
                                      isem[jp]).wait()
                pltpu.async_copy(g_hbm.at[cid[jp]], rows[jp], gsem[jp])
                load_dst(i + 2, jp)
            if j < 2:
                issue_next()
            else:
                pl.when(k < K - 1)(issue_next)

            pltpu.make_async_copy(dst_hbm.at[pl.ds(0, CH)], dst[j],
                                  dsem[j]).wait()
            pltpu.async_copy(rows[j], agg_sh.at[dst[j]], ssem[j], add=True)
        return carry

    lax.fori_loop(0, K, body, 0)
    for j in (2, 3):
        pltpu.make_async_copy(rows[j], agg_sh.at[dst[j]], ssem[j]).wait()
    plsc.subcore_barrier()
    NWB = RPT // CH

    def wb_load(j):
        pltpu.async_copy(agg_sh.at[pl.ds(s * RPT + j * CH, CH)], rows[j % 4],
                         gsem[j % 4])

    for j in range(4):
        wb_load(j)
    for j in range(NWB):
        b = j % 4
        pltpu.make_async_copy(agg_sh.at[pl.ds(0, CH)], rows[b],
                              gsem[b]).wait()
        pltpu.async_copy(rows[b],
                         out_hbm.at[pl.ds(c * NPAD + s * RPT + j * CH, CH)],
                         ssem[b])
        if j + 4 < NWB:
            pltpu.make_async_copy(rows[b], out_hbm.at[pl.ds(0, CH)],
                                  ssem[b]).wait()
            wb_load(j + 4)
    for j in range(NWB - 4, NWB):
        pltpu.make_async_copy(rows[j % 4], out_hbm.at[pl.ds(0, CH)],
                              ssem[j % 4]).wait()


@functools.cache
def _build_sc_agg():
    return pl.kernel(
        _sc_agg_body,
        out_type=jax.ShapeDtypeStruct((NC * NPAD, HID), _f32),
        mesh=plsc.VectorSubcoreMesh(core_axis_name="c", subcore_axis_name="s",
                                    num_cores=NC, num_subcores=NS),
        scratch_types=(
            [pltpu.VMEM((CH, HID), _f32)] * 4
            + [pltpu.VMEM((CH,), jnp.int32)] * 8
            + [pltpu.VMEM_SHARED((NPAD, HID), _f32)]
            + [pltpu.SemaphoreType.DMA] * 16
        ),
    )


def _sc_agg(g, cidx_p, dst_p, zeros_c):
    return _build_sc_agg()(g.reshape(4 * NPAD, HID), cidx_p, dst_p, zeros_c)


def _encode_body(xv_ref, atom_ref, bond_ref, h_ref, g_ref):
    ids = xv_ref[...]
    oh = (ids == lax.broadcasted_iota(jnp.int32, (BLK, 32), 1).astype(_f32)
          ).astype(_f32)
    h = jnp.dot(oh, atom_ref[...], preferred_element_type=_f32)
    h_ref[...] = h
    for t in range(4):
        g_ref[t] = jnp.maximum(h + bond_ref[t], 0.0)


def _encode(xvf, atom_p, bond_p):
    return pl.pallas_call(
        _encode_body,
        grid=(NBLK,),
        in_specs=[
            pl.BlockSpec((BLK, 1), lambda b: (b, 0)),
            pl.BlockSpec((32, HID), lambda b: (0, 0)),
            pl.BlockSpec((8, HID), lambda b: (0, 0)),
        ],
        out_specs=[
            pl.BlockSpec((BLK, HID), lambda b: (b, 0)),
            pl.BlockSpec((4, BLK, HID), lambda b: (0, b, 0)),
        ],
        out_shape=[
            jax.ShapeDtypeStruct((NPAD, HID), _f32),
            jax.ShapeDtypeStruct((4, NPAD, HID), _f32),
        ],
    )(xvf, atom_p, bond_p)


def _layer_body(h_ref, agg_ref, w1_ref, b1_ref, w2_ref, b2_ref, eps_ref,
                bond_ref, hout_ref, g_ref):
    pre = h_ref[...] * eps_ref[...] + agg_ref[0] + agg_ref[1]
    t = jnp.maximum(
        jnp.dot(pre, w1_ref[...], preferred_element_type=_f32) + b1_ref[...],
        0.0)
    out = jnp.dot(t, w2_ref[...], preferred_element_type=_f32) + b2_ref[...]
    hout_ref[...] = out
    for k in range(4):
        g_ref[k] = jnp.maximum(out + bond_ref[k], 0.0)


def _layer(h, agg2, w1, b1, w2, b2, epsr, bond_p):
    return pl.pallas_call(
        _layer_body,
        grid=(NBLK,),
        in_specs=[
            pl.BlockSpec((BLK, HID), lambda b: (b, 0)),
            pl.BlockSpec((2, BLK, HID), lambda b: (0, b, 0)),
            pl.BlockSpec((HID, HID), lambda b: (0, 0)),
            pl.BlockSpec((1, HID), lambda b: (0, 0)),
            pl.BlockSpec((HID, HID), lambda b: (0, 0)),
            pl.BlockSpec((1, HID), lambda b: (0, 0)),
            pl.BlockSpec((1, HID), lambda b: (0, 0)),
            pl.BlockSpec((8, HID), lambda b: (0, 0)),
        ],
        out_specs=[
            pl.BlockSpec((BLK, HID), lambda b: (b, 0)),
            pl.BlockSpec((4, BLK, HID), lambda b: (0, b, 0)),
        ],
        out_shape=[
            jax.ShapeDtypeStruct((NPAD, HID), _f32),
            jax.ShapeDtypeStruct((4, NPAD, HID), _f32),
        ],
    )(h, agg2, w1, b1, w2, b2, epsr, bond_p)


def _final_body(h_ref, agg_ref, w1_ref, b1_ref, w2_ref, b2_ref, eps_ref,
                bat_ref, fcw_ref, fcb_ref, pred_ref, sums, cnt):
    b = pl.program_id(0)
    pre = h_ref[...] * eps_ref[...] + agg_ref[0] + agg_ref[1]
    t = jnp.maximum(
        jnp.dot(pre, w1_ref[...], preferred_element_type=_f32) + b1_ref[...],
        0.0)
    out = jnp.dot(t, w2_ref[...], preferred_element_type=_f32) + b2_ref[...]
    oh = (bat_ref[...] ==
          lax.broadcasted_iota(jnp.int32, (BLK, HID), 1).astype(_f32)
          ).astype(_f32)
    part = lax.dot_general(oh, out, (((0,), (0,)), ((), ())),
                           preferred_element_type=_f32)
    cpart = lax.dot_general(oh, jnp.ones((BLK, HID), _f32),
                            (((0,), (0,)), ((), ())),
                            preferred_element_type=_f32)

    @pl.when(b == 0)
    def _():
        sums[...] = jnp.zeros_like(sums)
        cnt[...] = jnp.zeros_like(cnt)

    sums[...] += part
    cnt[...] += cpart

    @pl.when(b == NBLK - 1)
    def _():
        ge = sums[...] / jnp.maximum(cnt[...], 1.0)
        pred_ref[...] = (jnp.dot(ge, fcw_ref[...], preferred_element_type=_f32)
                         + fcb_ref[...])


def _final(h, agg2, w1, b1, w2, b2, epsr, batf, fcw, fcb):
    return pl.pallas_call(
        _final_body,
        grid=(NBLK,),
        in_specs=[
            pl.BlockSpec((BLK, HID), lambda b: (b, 0)),
            pl.BlockSpec((2, BLK, HID), lambda b: (0, b, 0)),
            pl.BlockSpec((HID, HID), lambda b: (0, 0)),
            pl.BlockSpec((1, HID), lambda b: (0, 0)),
            pl.BlockSpec((HID, HID), lambda b: (0, 0)),
            pl.BlockSpec((1, HID), lambda b: (0, 0)),
            pl.BlockSpec((1, HID), lambda b: (0, 0)),
            pl.BlockSpec((BLK, 1), lambda b: (b, 0)),
            pl.BlockSpec((HID, HID), lambda b: (0, 0)),
            pl.BlockSpec((1, HID), lambda b: (0, 0)),
        ],
        out_specs=pl.BlockSpec((HID, HID), lambda b: (0, 0)),
        out_shape=jax.ShapeDtypeStruct((HID, HID), _f32),
        scratch_shapes=[
            pltpu.VMEM((HID, HID), _f32),
            pltpu.VMEM((HID, HID), _f32),
        ],
    )(h, agg2, w1, b1, w2, b2, epsr, batf, fcw, fcb)


def kernel(x, edge_index, edge_attr, batch, atom_emb, bond_emb,
           conv_W1, conv_b1, conv_W2, conv_b2, conv_eps, fc_W, fc_b):
    xvf = jnp.concatenate(
        [x[:, 0].astype(_f32), jnp.full((NPAD - N,), -1.0, _f32)]
    ).reshape(NPAD, 1)
    src = edge_index[0].astype(jnp.int32)
    dst = edge_index[1].astype(jnp.int32)
    attr = edge_attr.astype(jnp.int32)
    padr = N + (jnp.arange(EPAD - E, dtype=jnp.int32) % (NPAD - N))
    cidx_p = jnp.concatenate([attr * NPAD + src, 3 * NPAD + padr])
    dst_p = jnp.concatenate([dst, padr])
    batf = jnp.concatenate(
        [batch.astype(_f32), jnp.full((NPAD - N,), 127.0, _f32)]
    ).reshape(NPAD, 1)
    atom_p = jnp.zeros((32, HID), _f32).at[:28].set(atom_emb)
    bond_p = jnp.zeros((8, HID), _f32).at[:4].set(bond_emb)
    zeros_c = jnp.zeros((CH, HID), _f32)
    b1 = conv_b1.reshape(NLAYERS, 1, HID)
    b2 = conv_b2.reshape(NLAYERS, 1, HID)
    epsr = jnp.broadcast_to((1.0 + conv_eps)[:, None, None],
                            (NLAYERS, 1, HID)).astype(_f32)
    fcb = fc_b.reshape(1, HID)

    h, g = _encode(xvf, atom_p, bond_p)
    for i in range(NLAYERS - 1):
        agg2 = _sc_agg(g, cidx_p, dst_p, zeros_c).reshape(NC, NPAD, HID)
        h, g = _layer(h, agg2, conv_W1[i], b1[i], conv_W2[i], b2[i],
                      epsr[i], bond_p)
    agg2 = _sc_agg(g, cidx_p, dst_p, zeros_c).reshape(NC, NPAD, HID)
    pred = _final(h, agg2, conv_W1[3], b1[3], conv_W2[3], b2[3],
                  epsr[3], batf, fc_W, fcb)
    return pred[:NGRAPHS]


# speedup vs baseline: 13.4446x; 1.0302x over previous
import functools

import jax
import jax.numpy as jnp
from jax import lax
from jax.experimental import pallas as pl
from jax.experimental.pallas import tpu as pltpu
from jax.experimental.pallas import tpu_sc as plsc

N = 10000
E = 320000
HID = 128
NLAYERS = 4
NGRAPHS = 100

NPAD = 10240
BLK = 1024
NBLK = NPAD // BLK

NC = 2
NS = 16
NW = NC * NS
CH = 80
NCHUNK = 128
EPW = NCHUNK * CH
EPAD = NW * EPW
RPT = NPAD // NS

_f32 = jnp.float32
_bf16 = jnp.bfloat16


def _sc_agg_body(g_hbm, cidx_hbm, dst_hbm, zeros_hbm, out_hbm, *sc):
    rows = sc[0:4]
    cid = sc[4:8]
    dst = sc[8:12]
    agg_sh = sc[12]
    gsem = sc[13:17]
    isem = sc[17:21]
    dsem = sc[21:25]
    ssem = sc[25:29]
    c = lax.axis_index("c")
    s = lax.axis_index("s")
    wid = s * NC + c
    ebase = wid * EPW
    K = NCHUNK // 4

    def load_cidx(chunk, j):
        pltpu.async_copy(cidx_hbm.at[pl.ds(ebase + chunk * CH, CH)], cid[j],
                         isem[j])

    def load_dst(chunk, j):
        pltpu.async_copy(dst_hbm.at[pl.ds(ebase + chunk * CH, CH)], dst[j],
                         dsem[j])

    pltpu.sync_copy(zeros_hbm, rows[0])
    for j in range(RPT // CH):
        pltpu.async_copy(rows[0], agg_sh.at[pl.ds(s * RPT + j * CH, CH)],
                         gsem[j % 4])
    for j in range(RPT // CH):
        pltpu.make_async_copy(
            rows[0], agg_sh.at[pl.ds(s * RPT + (j % 4) * CH, CH)],
            gsem[j % 4]).wait()
    plsc.subcore_barrier()

    for j in range(4):
        load_cidx(j, j)
    for j in range(2):
        load_dst(j, j)
    for j in range(2):
        pltpu.make_async_copy(cidx_hbm.at[pl.ds(0, CH)], cid[j], isem[j]).wait()
        pltpu.async_copy(g_hbm.at[cid[j]], rows[j], gsem[j])

    def body(k, carry):
        for j in range(4):
            jp = (j + 2) % 4
            i = 4 * k + j
            pltpu.make_async_copy(g_hbm.at[cid[j]], rows[j], gsem[j]).wait()

            @pl.when(k < K - 1)
            def _(i=i, j=j):
                load_cidx(i + 4, j)

            def wait_prev_scatter(jp=jp):
                pltpu.make_async_copy(rows[jp], agg_sh.at[dst[jp]],
                                      ssem[jp]).wait()
            if j < 2:
                pl.when(k > 0)(wait_prev_scatter)
            else:
                wait_prev_scatter()

            def issue_next(i=i, jp=jp):
                pltpu.make_async_copy(cidx_hbm.at[pl.ds(0, CH)], cid[jp],
                                      isem[jp]).wait()
                pltpu.async_copy(g_hbm.at[cid[jp]], rows[jp], gsem[jp])
                load_dst(i + 2, jp)
            if j < 2:
                issue_next()
            else:
                pl.when(k < K - 1)(issue_next)

            pltpu.make_async_copy(dst_hbm.at[pl.ds(0, CH)], dst[j],
                                  dsem[j]).wait()
            pltpu.async_copy(rows[j], agg_sh.at[dst[j]], ssem[j], add=True)
        return carry

    lax.fori_loop(0, K, body, 0)
    for j in (2, 3):
        pltpu.make_async_copy(rows[j], agg_sh.at[dst[j]], ssem[j]).wait()
    plsc.subcore_barrier()
    NWB = RPT // CH

    def wb_load(j):
        pltpu.async_copy(agg_sh.at[pl.ds(s * RPT + j * CH, CH)], rows[j % 4],
                         gsem[j % 4])

    for j in range(4):
        wb_load(j)
    for j in range(NWB):
        b = j % 4
        pltpu.make_async_copy(agg_sh.at[pl.ds(0, CH)], rows[b],
                              gsem[b]).wait()
        pltpu.async_copy(rows[b],
                         out_hbm.at[pl.ds(c * NPAD + s * RPT + j * CH, CH)],
                         ssem[b])
        if j + 4 < NWB:
            pltpu.make_async_copy(rows[b], out_hbm.at[pl.ds(0, CH)],
                                  ssem[b]).wait()
            wb_load(j + 4)
    for j in range(NWB - 4, NWB):
        pltpu.make_async_copy(rows[j % 4], out_hbm.at[pl.ds(0, CH)],
                              ssem[j % 4]).wait()


@functools.cache
def _build_sc_agg():
    return pl.kernel(
        _sc_agg_body,
        out_type=jax.ShapeDtypeStruct((NC * NPAD, HID), _f32),
        mesh=plsc.VectorSubcoreMesh(core_axis_name="c", subcore_axis_name="s",
                                    num_cores=NC, num_subcores=NS),
        scratch_types=(
            [pltpu.VMEM((CH, HID), _f32)] * 4
            + [pltpu.VMEM((CH,), jnp.int32)] * 8
            + [pltpu.VMEM_SHARED((NPAD, HID), _f32)]
            + [pltpu.SemaphoreType.DMA] * 16
        ),
    )


def _sc_agg(g, cidx_p, dst_p, zeros_c):
    return _build_sc_agg()(g.reshape(4 * NPAD, HID), cidx_p, dst_p, zeros_c)


def _encode_body(xv_ref, atom_ref, bond_ref, h_ref, g_ref):
    ids = xv_ref[...]
    oh = (ids == lax.broadcasted_iota(jnp.int32, (BLK, 32), 1).astype(_f32)
          ).astype(_f32)
    h = jnp.dot(oh, atom_ref[...], preferred_element_type=_f32)
    h_ref[...] = h
    for t in range(4):
        g_ref[t] = jnp.maximum(h + bond_ref[t], 0.0)


def _encode(xvf, atom_p, bond_p):
    return pl.pallas_call(
        _encode_body,
        grid=(NBLK,),
        in_specs=[
            pl.BlockSpec((BLK, 1), lambda b: (b, 0)),
            pl.BlockSpec((32, HID), lambda b: (0, 0)),
            pl.BlockSpec((8, HID), lambda b: (0, 0)),
        ],
        out_specs=[
            pl.BlockSpec((BLK, HID), lambda b: (b, 0)),
            pl.BlockSpec((4, BLK, HID), lambda b: (0, b, 0)),
        ],
        out_shape=[
            jax.ShapeDtypeStruct((NPAD, HID), _f32),
            jax.ShapeDtypeStruct((4, NPAD, HID), _f32),
        ],
    )(xvf, atom_p, bond_p)


def _layer_body(h_ref, agg_ref, w1_ref, b1_ref, w2_ref, b2_ref, eps_ref,
                bond_ref, hout_ref, g_ref):
    pre = h_ref[...] * eps_ref[...] + agg_ref[0] + agg_ref[1]
    t = jnp.maximum(
        jnp.dot(pre, w1_ref[...], preferred_element_type=_f32) + b1_ref[...],
        0.0)
    out = jnp.dot(t, w2_ref[...], preferred_element_type=_f32) + b2_ref[...]
    hout_ref[...] = out
    for k in range(4):
        g_ref[k] = jnp.maximum(out + bond_ref[k], 0.0)


def _layer(h, agg2, w1, b1, w2, b2, epsr, bond_p):
    return pl.pallas_call(
        _layer_body,
        grid=(NBLK,),
        in_specs=[
            pl.BlockSpec((BLK, HID), lambda b: (b, 0)),
            pl.BlockSpec((2, BLK, HID), lambda b: (0, b, 0)),
            pl.BlockSpec((HID, HID), lambda b: (0, 0)),
            pl.BlockSpec((1, HID), lambda b: (0, 0)),
            pl.BlockSpec((HID, HID), lambda b: (0, 0)),
            pl.BlockSpec((1, HID), lambda b: (0, 0)),
            pl.BlockSpec((1, HID), lambda b: (0, 0)),
            pl.BlockSpec((8, HID), lambda b: (0, 0)),
        ],
        out_specs=[
            pl.BlockSpec((BLK, HID), lambda b: (b, 0)),
            pl.BlockSpec((4, BLK, HID), lambda b: (0, b, 0)),
        ],
        out_shape=[
            jax.ShapeDtypeStruct((NPAD, HID), _f32),
            jax.ShapeDtypeStruct((4, NPAD, HID), _f32),
        ],
    )(h, agg2, w1, b1, w2, b2, epsr, bond_p)


def _final_body(h_ref, agg_ref, w1_ref, b1_ref, w2_ref, b2_ref, eps_ref,
                bat_ref, fcw_ref, fcb_ref, pred_ref, sums, cnt):
    b = pl.program_id(0)
    pre = h_ref[...] * eps_ref[...] + agg_ref[0] + agg_ref[1]
    t = jnp.maximum(
        jnp.dot(pre, w1_ref[...], preferred_element_type=_f32) + b1_ref[...],
        0.0)
    out = jnp.dot(t, w2_ref[...], preferred_element_type=_f32) + b2_ref[...]
    oh = (bat_ref[...] ==
          lax.broadcasted_iota(jnp.int32, (BLK, HID), 1).astype(_f32)
          ).astype(_f32)
    part = lax.dot_general(oh, out, (((0,), (0,)), ((), ())),
                           preferred_element_type=_f32)
    cpart = lax.dot_general(oh, jnp.ones((BLK, HID), _f32),
                            (((0,), (0,)), ((), ())),
                            preferred_element_type=_f32)

    @pl.when(b == 0)
    def _():
        sums[...] = jnp.zeros_like(sums)
        cnt[...] = jnp.zeros_like(cnt)

    sums[...] += part
    cnt[...] += cpart

    @pl.when(b == NBLK - 1)
    def _():
        ge = sums[...] / jnp.maximum(cnt[...], 1.0)
        pred_ref[...] = (jnp.dot(ge, fcw_ref[...], preferred_element_type=_f32)
                         + fcb_ref[...])


def _final(h, agg2, w1, b1, w2, b2, epsr, batf, fcw, fcb):
    return pl.pallas_call(
        _final_body,
        grid=(NBLK,),
        in_specs=[
            pl.BlockSpec((BLK, HID), lambda b: (b, 0)),
            pl.BlockSpec((2, BLK, HID), lambda b: (0, b, 0)),
            pl.BlockSpec((HID, HID), lambda b: (0, 0)),
            pl.BlockSpec((1, HID), lambda b: (0, 0)),
            pl.BlockSpec((HID, HID), lambda b: (0, 0)),
            pl.BlockSpec((1, HID), lambda b: (0, 0)),
            pl.BlockSpec((1, HID), lambda b: (0, 0)),
            pl.BlockSpec((BLK, 1), lambda b: (b, 0)),
            pl.BlockSpec((HID, HID), lambda b: (0, 0)),
            pl.BlockSpec((1, HID), lambda b: (0, 0)),
        ],
        out_specs=pl.BlockSpec((HID, HID), lambda b: (0, 0)),
        out_shape=jax.ShapeDtypeStruct((HID, HID), _f32),
        scratch_shapes=[
            pltpu.VMEM((HID, HID), _f32),
            pltpu.VMEM((HID, HID), _f32),
        ],
    )(h, agg2, w1, b1, w2, b2, epsr, batf, fcw, fcb)


def kernel(x, edge_index, edge_attr, batch, atom_emb, bond_emb,
           conv_W1, conv_b1, conv_W2, conv_b2, conv_eps, fc_W, fc_b):
    xvf = jnp.concatenate(
        [x[:, 0].astype(_f32), jnp.full((NPAD - N,), -1.0, _f32)]
    ).reshape(NPAD, 1)
    src = edge_index[0].astype(jnp.int32)
    dst = edge_index[1].astype(jnp.int32)
    attr = edge_attr.astype(jnp.int32)
    padr = N + (jnp.arange(EPAD - E, dtype=jnp.int32) % (NPAD - N))
    cidx_p = jnp.concatenate([attr * NPAD + src, 3 * NPAD + padr])
    dst_p = jnp.concatenate([dst, padr])
    batf = jnp.concatenate(
        [batch.astype(_f32), jnp.full((NPAD - N,), 127.0, _f32)]
    ).reshape(NPAD, 1)
    atom_p = jnp.zeros((32, HID), _f32).at[:28].set(atom_emb)
    bond_p = jnp.zeros((8, HID), _f32).at[:4].set(bond_emb)
    zeros_c = jnp.zeros((CH, HID), _f32)
    b1 = conv_b1.reshape(NLAYERS, 1, HID)
    b2 = conv_b2.reshape(NLAYERS, 1, HID)
    epsr = jnp.broadcast_to((1.0 + conv_eps)[:, None, None],
                            (NLAYERS, 1, HID)).astype(_f32)
    fcb = fc_b.reshape(1, HID)

    h, g = _encode(xvf, atom_p, bond_p)
    for i in range(NLAYERS - 1):
        agg2 = _sc_agg(g, cidx_p, dst_p, zeros_c).reshape(NC, NPAD, HID)
        h, g = _layer(h, agg2, conv_W1[i], b1[i], conv_W2[i], b2[i],
                      epsr[i], bond_p)
    agg2 = _sc_agg(g, cidx_p, dst_p, zeros_c).reshape(NC, NPAD, HID)
    pred = _final(h, agg2, conv_W1[3], b1[3], conv_W2[3], b2[3],
                  epsr[3], batf, fc_W, fcb)
    return pred[:NGRAPHS]


# speedup vs baseline: 13.4516x; 1.0005x over previous
import functools

import jax
import jax.numpy as jnp
from jax import lax
from jax.experimental import pallas as pl
from jax.experimental.pallas import tpu as pltpu
from jax.experimental.pallas import tpu_sc as plsc

N = 10000
E = 320000
HID = 128
NLAYERS = 4
NGRAPHS = 100

NPAD = 10240
BLK = 1024
NBLK = NPAD // BLK

NC = 2
NS = 16
NW = NC * NS
CH = 80
NCHUNK = 128
EPW = NCHUNK * CH
EPAD = NW * EPW
RPT = NPAD // NS

_f32 = jnp.float32


def _sc_agg_body(g_hbm, cidx_hbm, dst_hbm, zeros_hbm, out_hbm, *sc):
    rows = sc[0:4]
    cid = sc[4:8]
    dst = sc[8:12]
    agg_sh = sc[12]
    gsem = sc[13:17]
    isem = sc[17:21]
    dsem = sc[21:25]
    ssem = sc[25:29]
    c = lax.axis_index("c")
    s = lax.axis_index("s")
    wid = s * NC + c
    ebase = wid * EPW
    K = NCHUNK // 4

    def load_cidx(chunk, j):
        pltpu.async_copy(cidx_hbm.at[pl.ds(ebase + chunk * CH, CH)], cid[j],
                         isem[j])

    def load_dst(chunk, j):
        pltpu.async_copy(dst_hbm.at[pl.ds(ebase + chunk * CH, CH)], dst[j],
                         dsem[j])

    pltpu.sync_copy(zeros_hbm, rows[0])
    for j in range(RPT // CH):
        pltpu.async_copy(rows[0], agg_sh.at[pl.ds(s * RPT + j * CH, CH)],
                         gsem[j % 4])
    for j in range(RPT // CH):
        pltpu.make_async_copy(
            rows[0], agg_sh.at[pl.ds(s * RPT + (j % 4) * CH, CH)],
            gsem[j % 4]).wait()
    plsc.subcore_barrier()

    for j in range(4):
        load_cidx(j, j)
    for j in range(2):
        load_dst(j, j)
    for j in range(2):
        pltpu.make_async_copy(cidx_hbm.at[pl.ds(0, CH)], cid[j], isem[j]).wait()
        pltpu.async_copy(g_hbm.at[cid[j]], rows[j], gsem[j])

    def body(k, carry):
        for j in range(4):
            jp = (j + 2) % 4
            i = 4 * k + j
            pltpu.make_async_copy(g_hbm.at[cid[j]], rows[j], gsem[j]).wait()

            @pl.when(k < K - 1)
            def _(i=i, j=j):
                load_cidx(i + 4, j)

            def wait_prev_scatter(jp=jp):
                pltpu.make_async_copy(rows[jp], agg_sh.at[dst[jp]],
                                      ssem[jp]).wait()
            if j < 2:
                pl.when(k > 0)(wait_prev_scatter)
            else:
                wait_prev_scatter()

            def issue_next(i=i, jp=jp):
                pltpu.make_async_copy(cidx_hbm.at[pl.ds(0, CH)], cid[jp],
                                      isem[jp]).wait()
                pltpu.async_copy(g_hbm.at[cid[jp]], rows[jp], gsem[jp])
                load_dst(i + 2, jp)
            if j < 2:
                issue_next()
            else:
                pl.when(k < K - 1)(issue_next)

            pltpu.make_async_copy(dst_hbm.at[pl.ds(0, CH)], dst[j],
                                  dsem[j]).wait()
            pltpu.async_copy(rows[j], agg_sh.at[dst[j]], ssem[j], add=True)
        return carry

    lax.fori_loop(0, K, body, 0)
    for j in (2, 3):
        pltpu.make_async_copy(rows[j], agg_sh.at[dst[j]], ssem[j]).wait()
    plsc.subcore_barrier()
    NWB = RPT // CH

    def wb_load(j):
        pltpu.async_copy(agg_sh.at[pl.ds(s * RPT + j * CH, CH)], rows[j % 4],
                         gsem[j % 4])

    for j in range(4):
        wb_load(j)
    for j in range(NWB):
        b = j % 4
        pltpu.make_async_copy(agg_sh.at[pl.ds(0, CH)], rows[b],
                              gsem[b]).wait()
        pltpu.async_copy(rows[b],
                         out_hbm.at[pl.ds(c * NPAD + s * RPT + j * CH, CH)],
                         ssem[b])
        if j + 4 < NWB:
            pltpu.make_async_copy(rows[b], out_hbm.at[pl.ds(0, CH)],
                                  ssem[b]).wait()
            wb_load(j + 4)
    for j in range(NWB - 4, NWB):
        pltpu.make_async_copy(rows[j % 4], out_hbm.at[pl.ds(0, CH)],
                              ssem[j % 4]).wait()


@functools.cache
def _build_sc_agg():
    return pl.kernel(
        _sc_agg_body,
        out_type=jax.ShapeDtypeStruct((NC * NPAD, HID), _f32),
        mesh=plsc.VectorSubcoreMesh(core_axis_name="c", subcore_axis_name="s",
                                    num_cores=NC, num_subcores=NS),
        scratch_types=(
            [pltpu.VMEM((CH, HID), _f32)] * 4
            + [pltpu.VMEM((CH,), jnp.int32)] * 8
            + [pltpu.VMEM_SHARED((NPAD, HID), _f32)]
            + [pltpu.SemaphoreType.DMA] * 16
        ),
    )


def _sc_agg(g, cidx_p, dst_p, zeros_c):
    return _build_sc_agg()(g.reshape(4 * NPAD, HID), cidx_p, dst_p, zeros_c)


def _encode_body(xv_ref, atom_ref, bond_ref, h_ref, g_ref):
    ids = xv_ref[...]
    oh = (ids == lax.broadcasted_iota(jnp.int32, (BLK, 32), 1).astype(_f32)
          ).astype(_f32)
    h = jnp.dot(oh, atom_ref[...], preferred_element_type=_f32)
    h_ref[...] = h
    for t in range(4):
        g_ref[t] = jnp.maximum(h + bond_ref[t], 0.0)


def _encode(xvf, atom_p, bond_p):
    return pl.pallas_call(
        _encode_body,
        grid=(NBLK,),
        in_specs=[
            pl.BlockSpec((BLK, 1), lambda b: (b, 0)),
            pl.BlockSpec((32, HID), lambda b: (0, 0)),
            pl.BlockSpec((8, HID), lambda b: (0, 0)),
        ],
        out_specs=[
            pl.BlockSpec((BLK, HID), lambda b: (b, 0)),
            pl.BlockSpec((4, BLK, HID), lambda b: (0, b, 0)),
        ],
        out_shape=[
            jax.ShapeDtypeStruct((NPAD, HID), _f32),
            jax.ShapeDtypeStruct((4, NPAD, HID), _f32),
        ],
    )(xvf, atom_p, bond_p)


def _layer_body(h_ref, agg_ref, w1_ref, b1_ref, w2_ref, b2_ref, eps_ref,
                bond_ref, hout_ref, g_ref):
    pre = h_ref[...] * eps_ref[...] + agg_ref[0] + agg_ref[1]
    t = jnp.maximum(
        jnp.dot(pre, w1_ref[...], preferred_element_type=_f32) + b1_ref[...],
        0.0)
    out = jnp.dot(t, w2_ref[...], preferred_element_type=_f32) + b2_ref[...]
    hout_ref[...] = out
    for k in range(4):
        g_ref[k] = jnp.maximum(out + bond_ref[k], 0.0)


def _layer(h, agg2, w1, b1, w2, b2, epsr, bond_p):
    return pl.pallas_call(
        _layer_body,
        grid=(NBLK,),
        in_specs=[
            pl.BlockSpec((BLK, HID), lambda b: (b, 0)),
            pl.BlockSpec((2, BLK, HID), lambda b: (0, b, 0)),
            pl.BlockSpec((HID, HID), lambda b: (0, 0)),
            pl.BlockSpec((1, HID), lambda b: (0, 0)),
            pl.BlockSpec((HID, HID), lambda b: (0, 0)),
            pl.BlockSpec((1, HID), lambda b: (0, 0)),
            pl.BlockSpec((1, HID), lambda b: (0, 0)),
            pl.BlockSpec((8, HID), lambda b: (0, 0)),
        ],
        out_specs=[
            pl.BlockSpec((BLK, HID), lambda b: (b, 0)),
            pl.BlockSpec((4, BLK, HID), lambda b: (0, b, 0)),
        ],
        out_shape=[
            jax.ShapeDtypeStruct((NPAD, HID), _f32),
            jax.ShapeDtypeStruct((4, NPAD, HID), _f32),
        ],
    )(h, agg2, w1, b1, w2, b2, epsr, bond_p)


def _final_body(h_ref, agg_ref, w1_ref, b1_ref, w2_ref, b2_ref, eps_ref,
                bat_ref, fcw_ref, fcb_ref, pred_ref, sums, cnt):
    b = pl.program_id(0)
    pre = h_ref[...] * eps_ref[...] + agg_ref[0] + agg_ref[1]
    t = jnp.maximum(
        jnp.dot(pre, w1_ref[...], preferred_element_type=_f32) + b1_ref[...],
        0.0)
    out = jnp.dot(t, w2_ref[...], preferred_element_type=_f32) + b2_ref[...]
    oh = (bat_ref[...] ==
          lax.broadcasted_iota(jnp.int32, (BLK, HID), 1).astype(_f32)
          ).astype(_f32)
    part = lax.dot_general(oh, out, (((0,), (0,)), ((), ())),
                           preferred_element_type=_f32)
    cpart = lax.dot_general(oh, jnp.ones((BLK, HID), _f32),
                            (((0,), (0,)), ((), ())),
                            preferred_element_type=_f32)

    @pl.when(b == 0)
    def _():
        sums[...] = jnp.zeros_like(sums)
        cnt[...] = jnp.zeros_like(cnt)

    sums[...] += part
    cnt[...] += cpart

    @pl.when(b == NBLK - 1)
    def _():
        ge = sums[...] / jnp.maximum(cnt[...], 1.0)
        pred_ref[...] = (jnp.dot(ge, fcw_ref[...], preferred_element_type=_f32)
                         + fcb_ref[...])


def _final(h, agg2, w1, b1, w2, b2, epsr, batf, fcw, fcb):
    return pl.pallas_call(
        _final_body,
        grid=(NBLK,),
        in_specs=[
            pl.BlockSpec((BLK, HID), lambda b: (b, 0)),
            pl.BlockSpec((2, BLK, HID), lambda b: (0, b, 0)),
            pl.BlockSpec((HID, HID), lambda b: (0, 0)),
            pl.BlockSpec((1, HID), lambda b: (0, 0)),
            pl.BlockSpec((HID, HID), lambda b: (0, 0)),
            pl.BlockSpec((1, HID), lambda b: (0, 0)),
            pl.BlockSpec((1, HID), lambda b: (0, 0)),
            pl.BlockSpec((BLK, 1), lambda b: (b, 0)),
            pl.BlockSpec((HID, HID), lambda b: (0, 0)),
            pl.BlockSpec((1, HID), lambda b: (0, 0)),
        ],
        out_specs=pl.BlockSpec((HID, HID), lambda b: (0, 0)),
        out_shape=jax.ShapeDtypeStruct((HID, HID), _f32),
        scratch_shapes=[
            pltpu.VMEM((HID, HID), _f32),
            pltpu.VMEM((HID, HID), _f32),
        ],
    )(h, agg2, w1, b1, w2, b2, epsr, batf, fcw, fcb)


def kernel(x, edge_index, edge_attr, batch, atom_emb, bond_emb,
           conv_W1, conv_b1, conv_W2, conv_b2, conv_eps, fc_W, fc_b):
    xvf = jnp.concatenate(
        [x[:, 0].astype(_f32), jnp.full((NPAD - N,), -1.0, _f32)]
    ).reshape(NPAD, 1)
    src = edge_index[0].astype(jnp.int32)
    dst = edge_index[1].astype(jnp.int32)
    attr = edge_attr.astype(jnp.int32)
    padr = N + (jnp.arange(EPAD - E, dtype=jnp.int32) % (NPAD - N))
    cidx_p = jnp.concatenate([attr * NPAD + src, 3 * NPAD + padr])
    dst_p = jnp.concatenate([dst, padr])
    batf = jnp.concatenate(
        [batch.astype(_f32), jnp.full((NPAD - N,), 127.0, _f32)]
    ).reshape(NPAD, 1)
    atom_p = jnp.zeros((32, HID), _f32).at[:28].set(atom_emb)
    bond_p = jnp.zeros((8, HID), _f32).at[:4].set(bond_emb)
    zeros_c = jnp.zeros((CH, HID), _f32)
    b1 = conv_b1.reshape(NLAYERS, 1, HID)
    b2 = conv_b2.reshape(NLAYERS, 1, HID)
    epsr = jnp.broadcast_to((1.0 + conv_eps)[:, None, None],
                            (NLAYERS, 1, HID)).astype(_f32)
    fcb = fc_b.reshape(1, HID)

    h, g = _encode(xvf, atom_p, bond_p)
    for i in range(NLAYERS - 1):
        agg2 = _sc_agg(g, cidx_p, dst_p, zeros_c).reshape(NC, NPAD, HID)
        h, g = _layer(h, agg2, conv_W1[i], b1[i], conv_W2[i], b2[i],
                      epsr[i], bond_p)
    agg2 = _sc_agg(g, cidx_p, dst_p, zeros_c).reshape(NC, NPAD, HID)
    pred = _final(h, agg2, conv_W1[3], b1[3], conv_W2[3], b2[3],
                  epsr[3], batf, fc_W, fcb)
    return pred[:NGRAPHS]
